# Initial kernel scaffold; baseline (speedup 1.0000x reference)
import jax
import jax.numpy as jnp
from jax.experimental import pallas as pl


def kernel(x_query, e_p, e_k, vis_mark):
    raise NotImplementedError("write your pallas kernel here")


# speedup vs baseline: 1.1395x; 1.1395x over previous
import functools

import jax
import jax.numpy as jnp
from jax import lax
from jax.experimental import pallas as pl
from jax.experimental.pallas import tpu as pltpu
from jax.experimental.pallas import tpu_sc as plsc

EMB_D = 768
KEY_D = 768
NUM_LAYERS = 12
POOL = 512
P_LEN = 8
TOPK = 5
B = 128

ROW_D = P_LEN * EMB_D
N_ROWS = NUM_LAYERS * B * TOPK
N_TABLE = NUM_LAYERS * POOL

NC = 2
NS = 16
NW = NC * NS
ROWS_PER_W = N_ROWS // NW
CHUNK = 8
N_CHUNKS = ROWS_PER_W // CHUNK


def _topk_body(x_ref, ek_ref, idx_ref):
    q = x_ref[0]
    ek = ek_ref[0]
    qn = q / jnp.maximum(jnp.sqrt(jnp.sum(q * q, axis=1, keepdims=True)), 1e-12)
    kn = ek / jnp.maximum(jnp.sqrt(jnp.sum(ek * ek, axis=1, keepdims=True)), 1e-12)
    cos = lax.dot_general(qn, kn, (((1,), (1,)), ((), ())),
                          preferred_element_type=jnp.float32)
    l = pl.program_id(0)
    iota = lax.broadcasted_iota(jnp.int32, (B, POOL), 1)
    offset = l * POOL
    work = cos
    cols = []
    for _ in range(TOPK):
        m = jnp.max(work, axis=1, keepdims=True)
        idx_t = jnp.min(jnp.where(work == m, iota, POOL), axis=1)
        work = jnp.where(iota == idx_t[:, None], -jnp.inf, work)
        cols.append(idx_t + offset)
    idx_ref[0] = jnp.stack(cols, axis=1)


def _topk_indices(x_query, e_k):
    return pl.pallas_call(
        _topk_body,
        grid=(NUM_LAYERS,),
        in_specs=[
            pl.BlockSpec((1, B, KEY_D), lambda l: (l, 0, 0)),
            pl.BlockSpec((1, POOL, KEY_D), lambda l: (l, 0, 0)),
        ],
        out_specs=pl.BlockSpec((1, B, TOPK), lambda l: (l, 0, 0)),
        out_shape=jax.ShapeDtypeStruct((NUM_LAYERS, B, TOPK), jnp.int32),
    )(x_query, e_k)


def _sc_gather_body(table_hbm, idx_hbm, out_hbm, idx_v, buf, sem0, sem1):
    wid = lax.axis_index("s") * NC + lax.axis_index("c")
    base = wid * ROWS_PER_W
    pltpu.sync_copy(idx_hbm.at[pl.ds(base, ROWS_PER_W)], idx_v)
    sems = (sem0, sem1)

    def start(c, b):
        off = pl.multiple_of(c * CHUNK, CHUNK)
        pltpu.make_async_copy(
            table_hbm.at[idx_v.at[pl.ds(off, CHUNK)]], buf.at[b], sems[b]
        ).start()

    def finish(c, b):
        pltpu.make_async_copy(
            table_hbm.at[idx_v.at[pl.ds(0, CHUNK)]], buf.at[b], sems[b]
        ).wait()
        pltpu.sync_copy(buf.at[b], out_hbm.at[pl.ds(base + c * CHUNK, CHUNK)])

    start(0, 0)
    start(1, 1)

    def body(i, _):
        for b in range(2):
            c = 2 * i + b
            finish(c, b)

            @pl.when(c + 2 < N_CHUNKS)
            def _():
                start(c + 2, b)
        return _

    lax.fori_loop(0, N_CHUNKS // 2, body, None)


def _sc_gather(table, flat_idx):
    mesh = plsc.VectorSubcoreMesh(core_axis_name="c", subcore_axis_name="s")
    return pl.kernel(
        _sc_gather_body,
        out_type=jax.ShapeDtypeStruct((N_ROWS, ROW_D), jnp.float32),
        mesh=mesh,
        scratch_types=[
            pltpu.VMEM((ROWS_PER_W,), jnp.int32),
            pltpu.VMEM((2, CHUNK, ROW_D), jnp.float32),
            pltpu.SemaphoreType.DMA,
            pltpu.SemaphoreType.DMA,
        ],
    )(table, flat_idx)


def kernel(x_query, e_p, e_k, vis_mark):
    del vis_mark
    g_idx = _topk_indices(jnp.transpose(x_query, (1, 0, 2)), e_k)
    flat_idx = g_idx.reshape(N_ROWS)
    table = e_p.reshape(N_TABLE, ROW_D)
    rows = _sc_gather(table, flat_idx)
    p_return = rows.reshape(NUM_LAYERS, B, TOPK * P_LEN, EMB_D)
    return (p_return, jnp.float32(0.0))


# speedup vs baseline: 2.9239x; 2.5659x over previous
import functools

import jax
import jax.numpy as jnp
from jax import lax
from jax.experimental import pallas as pl
from jax.experimental.pallas import tpu as pltpu
from jax.experimental.pallas import tpu_sc as plsc

EMB_D = 768
KEY_D = 768
NUM_LAYERS = 12
POOL = 512
P_LEN = 8
TOPK = 5
B = 128

ROW_D = P_LEN * EMB_D
N_ROWS = NUM_LAYERS * B * TOPK
N_TABLE = NUM_LAYERS * POOL

NC = 2
NS = 16
NW = NC * NS
ROWS_PER_W = N_ROWS // NW
CHUNK = 8
N_CHUNKS = ROWS_PER_W // CHUNK


def _topk_body(x_ref, ek_ref, idx_ref):
    q = x_ref[0]
    ek = ek_ref[0]
    qn = q / jnp.maximum(jnp.sqrt(jnp.sum(q * q, axis=1, keepdims=True)), 1e-12)
    kn = ek / jnp.maximum(jnp.sqrt(jnp.sum(ek * ek, axis=1, keepdims=True)), 1e-12)
    cos = lax.dot_general(qn, kn, (((1,), (1,)), ((), ())),
                          preferred_element_type=jnp.float32)
    l = pl.program_id(0)
    iota = lax.broadcasted_iota(jnp.int32, (B, POOL), 1)
    offset = l * POOL
    work = cos
    cols = []
    for _ in range(TOPK):
        m = jnp.max(work, axis=1, keepdims=True)
        idx_t = jnp.min(jnp.where(work == m, iota, POOL), axis=1)
        work = jnp.where(iota == idx_t[:, None], -jnp.inf, work)
        cols.append(idx_t + offset)
    idx_ref[0] = jnp.stack(cols, axis=1)


def _topk_indices(x_query, e_k):
    return pl.pallas_call(
        _topk_body,
        grid=(NUM_LAYERS,),
        in_specs=[
            pl.BlockSpec((1, B, KEY_D), lambda l: (l, 0, 0)),
            pl.BlockSpec((1, POOL, KEY_D), lambda l: (l, 0, 0)),
        ],
        out_specs=pl.BlockSpec((1, B, TOPK), lambda l: (l, 0, 0)),
        out_shape=jax.ShapeDtypeStruct((NUM_LAYERS, B, TOPK), jnp.int32),
    )(x_query, e_k)


def _sc_gather_body(table_hbm, idx_hbm, out_hbm, idx_v, buf, sem0, sem1):
    wid = lax.axis_index("s") * NC + lax.axis_index("c")
    base = wid * ROWS_PER_W
    pltpu.sync_copy(idx_hbm.at[pl.ds(base, ROWS_PER_W)], idx_v)
    sems = (sem0, sem1)

    def start(c, b):
        off = pl.multiple_of(c * CHUNK, CHUNK)
        pltpu.make_async_copy(
            table_hbm.at[idx_v.at[pl.ds(off, CHUNK)]], buf.at[b], sems[b]
        ).start()

    def finish(c, b):
        pltpu.make_async_copy(
            table_hbm.at[idx_v.at[pl.ds(0, CHUNK)]], buf.at[b], sems[b]
        ).wait()
        pltpu.sync_copy(buf.at[b], out_hbm.at[pl.ds(base + c * CHUNK, CHUNK)])

    start(0, 0)
    start(1, 1)

    def body(i, _):
        for b in range(2):
            c = 2 * i + b
            finish(c, b)

            @pl.when(c + 2 < N_CHUNKS)
            def _():
                start(c + 2, b)
        return _

    lax.fori_loop(0, N_CHUNKS // 2, body, None)


def _sc_gather(table, flat_idx):
    mesh = plsc.VectorSubcoreMesh(core_axis_name="c", subcore_axis_name="s")
    return pl.kernel(
        _sc_gather_body,
        out_type=jax.ShapeDtypeStruct((N_ROWS, P_LEN, EMB_D), jnp.float32),
        mesh=mesh,
        scratch_types=[
            pltpu.VMEM((ROWS_PER_W,), jnp.int32),
            pltpu.VMEM((2, CHUNK, P_LEN, EMB_D), jnp.float32),
            pltpu.SemaphoreType.DMA,
            pltpu.SemaphoreType.DMA,
        ],
    )(table, flat_idx)


def kernel(x_query, e_p, e_k, vis_mark):
    del vis_mark
    g_idx = _topk_indices(jnp.transpose(x_query, (1, 0, 2)), e_k)
    flat_idx = g_idx.reshape(N_ROWS)
    table = e_p.reshape(N_TABLE, P_LEN, EMB_D)
    rows = _sc_gather(table, flat_idx)
    p_return = rows.reshape(NUM_LAYERS, B, TOPK * P_LEN, EMB_D)
    return (p_return, jnp.float32(0.0))


# speedup vs baseline: 2.9305x; 1.0023x over previous
import functools

import jax
import jax.numpy as jnp
from jax import lax
from jax.experimental import pallas as pl
from jax.experimental.pallas import tpu as pltpu
from jax.experimental.pallas import tpu_sc as plsc

EMB_D = 768
KEY_D = 768
NUM_LAYERS = 12
POOL = 512
P_LEN = 8
TOPK = 5
B = 128

ROW_D = P_LEN * EMB_D
N_ROWS = NUM_LAYERS * B * TOPK
N_TABLE = NUM_LAYERS * POOL

NC = 2
NS = 16
NW = NC * NS
ROWS_PER_W = N_ROWS // NW
CHUNK = 8
N_CHUNKS = ROWS_PER_W // CHUNK


def _topk_body(x_ref, ek_ref, idx_ref):
    q = x_ref[0]
    ek = ek_ref[0]
    qn = q / jnp.maximum(jnp.sqrt(jnp.sum(q * q, axis=1, keepdims=True)), 1e-12)
    kn = ek / jnp.maximum(jnp.sqrt(jnp.sum(ek * ek, axis=1, keepdims=True)), 1e-12)
    cos = lax.dot_general(qn, kn, (((1,), (1,)), ((), ())),
                          preferred_element_type=jnp.float32)
    l = pl.program_id(0)
    iota = lax.broadcasted_iota(jnp.int32, (B, POOL), 1)
    offset = l * POOL
    work = cos
    cols = []
    for _ in range(TOPK):
        m = jnp.max(work, axis=1, keepdims=True)
        idx_t = jnp.min(jnp.where(work == m, iota, POOL), axis=1)
        work = jnp.where(iota == idx_t[:, None], -jnp.inf, work)
        cols.append(idx_t + offset)
    idx_ref[0] = jnp.stack(cols, axis=1)


def _topk_indices(x_query, e_k):
    return pl.pallas_call(
        _topk_body,
        grid=(NUM_LAYERS,),
        in_specs=[
            pl.BlockSpec((1, B, KEY_D), lambda l: (l, 0, 0)),
            pl.BlockSpec((1, POOL, KEY_D), lambda l: (l, 0, 0)),
        ],
        out_specs=pl.BlockSpec((1, B, TOPK), lambda l: (l, 0, 0)),
        out_shape=jax.ShapeDtypeStruct((NUM_LAYERS, B, TOPK), jnp.int32),
    )(x_query, e_k)


def _sc_gather_body(table_hbm, idx_hbm, out_hbm, idx_v, buf,
                    gsem0, gsem1, wsem0, wsem1):
    wid = lax.axis_index("s") * NC + lax.axis_index("c")
    base = wid * ROWS_PER_W
    pltpu.sync_copy(idx_hbm.at[pl.ds(base, ROWS_PER_W)], idx_v)
    gsems = (gsem0, gsem1)
    wsems = (wsem0, wsem1)

    def start_gather(c, b):
        off = pl.multiple_of(c * CHUNK, CHUNK)
        pltpu.make_async_copy(
            table_hbm.at[idx_v.at[pl.ds(off, CHUNK)]], buf.at[b], gsems[b]
        ).start()

    def wait_gather(b):
        pltpu.make_async_copy(
            table_hbm.at[idx_v.at[pl.ds(0, CHUNK)]], buf.at[b], gsems[b]
        ).wait()

    def start_write(c, b):
        pltpu.make_async_copy(
            buf.at[b], out_hbm.at[pl.ds(base + c * CHUNK, CHUNK)], wsems[b]
        ).start()

    def wait_write(b):
        pltpu.make_async_copy(
            buf.at[b], out_hbm.at[pl.ds(base, CHUNK)], wsems[b]
        ).wait()

    start_gather(0, 0)
    start_gather(1, 1)

    def body(i, _):
        for b in range(2):
            c = 2 * i + b
            wait_gather(b)
            start_write(c, b)

            @pl.when(c + 2 < N_CHUNKS)
            def _():
                wait_write(b)
                start_gather(c + 2, b)
        return _

    lax.fori_loop(0, N_CHUNKS // 2, body, None)
    wait_write(0)
    wait_write(1)


def _sc_gather(table, flat_idx):
    mesh = plsc.VectorSubcoreMesh(core_axis_name="c", subcore_axis_name="s")
    return pl.kernel(
        _sc_gather_body,
        out_type=jax.ShapeDtypeStruct((N_ROWS, P_LEN, EMB_D), jnp.float32),
        mesh=mesh,
        scratch_types=[
            pltpu.VMEM((ROWS_PER_W,), jnp.int32),
            pltpu.VMEM((2, CHUNK, P_LEN, EMB_D), jnp.float32),
            pltpu.SemaphoreType.DMA,
            pltpu.SemaphoreType.DMA,
            pltpu.SemaphoreType.DMA,
            pltpu.SemaphoreType.DMA,
        ],
    )(table, flat_idx)


def kernel(x_query, e_p, e_k, vis_mark):
    del vis_mark
    g_idx = _topk_indices(jnp.transpose(x_query, (1, 0, 2)), e_k)
    flat_idx = g_idx.reshape(N_ROWS)
    table = e_p.reshape(N_TABLE, P_LEN, EMB_D)
    rows = _sc_gather(table, flat_idx)
    p_return = rows.reshape(NUM_LAYERS, B, TOPK * P_LEN, EMB_D)
    return (p_return, jnp.float32(0.0))


# speedup vs baseline: 2.9766x; 1.0157x over previous
import functools

import jax
import jax.numpy as jnp
from jax import lax
from jax.experimental import pallas as pl
from jax.experimental.pallas import tpu as pltpu
from jax.experimental.pallas import tpu_sc as plsc

EMB_D = 768
KEY_D = 768
NUM_LAYERS = 12
POOL = 512
P_LEN = 8
TOPK = 5
B = 128

ROW_D = P_LEN * EMB_D
N_ROWS = NUM_LAYERS * B * TOPK
N_TABLE = NUM_LAYERS * POOL

NC = 2
NS = 16
NW = NC * NS
ROWS_PER_W = N_ROWS // NW
CHUNK = TOPK
IDX_PAD = 8
NBUF = 4
N_CHUNKS = ROWS_PER_W // CHUNK
TOT_CHUNKS = N_ROWS // CHUNK


def _topk_body(x_ref, ek_ref, idx_ref):
    q = x_ref[0]
    ek = ek_ref[0]
    qn = q / jnp.maximum(jnp.sqrt(jnp.sum(q * q, axis=1, keepdims=True)), 1e-12)
    kn = ek / jnp.maximum(jnp.sqrt(jnp.sum(ek * ek, axis=1, keepdims=True)), 1e-12)
    cos = lax.dot_general(qn, kn, (((1,), (1,)), ((), ())),
                          preferred_element_type=jnp.float32)
    l = pl.program_id(0)
    iota = lax.broadcasted_iota(jnp.int32, (B, POOL), 1)
    offset = l * POOL
    work = cos
    cols = []
    for _ in range(TOPK):
        m = jnp.max(work, axis=1, keepdims=True)
        idx_t = jnp.min(jnp.where(work == m, iota, POOL), axis=1)
        work = jnp.where(iota == idx_t[:, None], -jnp.inf, work)
        cols.append(idx_t + offset)
    cols += [jnp.zeros((B,), jnp.int32)] * (IDX_PAD - TOPK)
    idx_ref[...] = jnp.stack(cols, axis=1)


def _topk_indices(x_query, e_k):
    return pl.pallas_call(
        _topk_body,
        grid=(NUM_LAYERS,),
        in_specs=[
            pl.BlockSpec((1, B, KEY_D), lambda l: (l, 0, 0)),
            pl.BlockSpec((1, POOL, KEY_D), lambda l: (l, 0, 0)),
        ],
        out_specs=pl.BlockSpec((B, IDX_PAD), lambda l: (l, 0)),
        out_shape=jax.ShapeDtypeStruct((TOT_CHUNKS, IDX_PAD), jnp.int32),
    )(x_query, e_k)


def _sc_gather_body(table_hbm, idx_hbm, out_hbm, idx_v, buf, *sems):
    gsems = sems[:NBUF]
    wsems = sems[NBUF:]
    wid = lax.axis_index("s") * NC + lax.axis_index("c")
    base = wid * ROWS_PER_W
    pltpu.sync_copy(idx_hbm.at[pl.ds(wid * N_CHUNKS, N_CHUNKS)], idx_v)

    def start_gather(c, b):
        pltpu.make_async_copy(
            table_hbm.at[idx_v.at[c, pl.ds(0, CHUNK)]], buf.at[b], gsems[b]
        ).start()

    def wait_gather(b):
        pltpu.make_async_copy(
            table_hbm.at[idx_v.at[0, pl.ds(0, CHUNK)]], buf.at[b], gsems[b]
        ).wait()

    def start_write(c, b):
        pltpu.make_async_copy(
            buf.at[b], out_hbm.at[pl.ds(base + c * CHUNK, CHUNK)], wsems[b]
        ).start()

    def wait_write(b):
        pltpu.make_async_copy(
            buf.at[b], out_hbm.at[pl.ds(base, CHUNK)], wsems[b]
        ).wait()

    for b in range(NBUF):
        start_gather(b, b)

    def body(i, _):
        for b in range(NBUF):
            c = NBUF * i + b
            wait_gather(b)
            start_write(c, b)

            @pl.when(c + NBUF < N_CHUNKS)
            def _():
                wait_write(b)
                start_gather(c + NBUF, b)
        return _

    lax.fori_loop(0, N_CHUNKS // NBUF, body, None)
    for b in range(NBUF):
        wait_write(b)


def _sc_gather(table, pad_idx):
    mesh = plsc.VectorSubcoreMesh(core_axis_name="c", subcore_axis_name="s")
    return pl.kernel(
        _sc_gather_body,
        out_type=jax.ShapeDtypeStruct((N_ROWS, P_LEN, EMB_D), jnp.float32),
        mesh=mesh,
        scratch_types=[
            pltpu.VMEM((N_CHUNKS, IDX_PAD), jnp.int32),
            pltpu.VMEM((NBUF, CHUNK, P_LEN, EMB_D), jnp.float32),
        ] + [pltpu.SemaphoreType.DMA] * (2 * NBUF),
    )(table, pad_idx)


def kernel(x_query, e_p, e_k, vis_mark):
    del vis_mark
    pad_idx = _topk_indices(jnp.transpose(x_query, (1, 0, 2)), e_k)
    table = e_p.reshape(N_TABLE, P_LEN, EMB_D)
    rows = _sc_gather(table, pad_idx)
    p_return = rows.reshape(NUM_LAYERS, B, TOPK * P_LEN, EMB_D)
    return (p_return, jnp.float32(0.0))
